# Initial kernel scaffold; baseline (speedup 1.0000x reference)
import jax
import jax.numpy as jnp
from jax.experimental import pallas as pl


def kernel(user_emb, item_emb, adj_val, adj_row, adj_col):
    raise NotImplementedError("write your pallas kernel here")


# speedup vs baseline: 2.8511x; 2.8511x over previous
import functools
import jax
import jax.numpy as jnp
from jax import lax
from jax.experimental import pallas as pl
from jax.experimental.pallas import tpu as pltpu, tpu_sc as plsc

_USER_NUM = 20000
_ITEM_NUM = 30000
_N = _USER_NUM + _ITEM_NUM
_E = 800000
_EMB = 64

_HALF = _N // 2
_NTILES = 16
_CHUNK = 128
_EDGES_PER_TILE = _E // _NTILES
_NFULL = _EDGES_PER_TILE // _CHUNK
_REM = _EDGES_PER_TILE - _NFULL * _CHUNK
_ACC_ROWS = 25088
_ZROWS = _ACC_ROWS // _NTILES
_CP_ROWS = 1568

_mesh = plsc.VectorSubcoreMesh(core_axis_name="c", subcore_axis_name="s")


@functools.partial(
    pl.kernel,
    mesh=_mesh,
    compiler_params=pltpu.CompilerParams(use_tc_tiling_on_sc=False),
    out_type=jax.ShapeDtypeStruct((_N, _EMB), jnp.float32),
    scratch_types=[
        pltpu.VMEM((_CHUNK,), jnp.int32),
        pltpu.VMEM((_CHUNK,), jnp.int32),
        pltpu.VMEM((_CHUNK,), jnp.int32),
        pltpu.VMEM((_CHUNK,), jnp.float32),
        pltpu.VMEM((_CHUNK, _EMB), jnp.float32),
        pltpu.VMEM_SHARED((_ACC_ROWS, _EMB), jnp.float32),
        pltpu.SemaphoreType.DMA,
    ],
)
def _spmm(x_hbm, row_hbm, col_hbm, val_hbm, y_hbm,
          colb, rowb, idxb, valb, rowsb, acc, sem):
    c = lax.axis_index("c")
    s = lax.axis_index("s")
    lo = c * _HALF
    dump = _HALF + s

    zero16 = jnp.zeros((16,), jnp.float32)

    def _zero_rowsb(r, carry):
        for q in range(_EMB // 16):
            rowsb[r, pl.ds(q * 16, 16)] = zero16
        return carry

    lax.fori_loop(0, _CHUNK, _zero_rowsb, 0)
    zbase = s * _ZROWS
    for k in range(_ZROWS // _CHUNK):
        pltpu.sync_copy(rowsb, acc.at[pl.ds(zbase + k * _CHUNK, _CHUNK)])
    pltpu.sync_copy(rowsb, acc.at[pl.ds(zbase + _ZROWS - _CHUNK, _CHUNK)])
    plsc.subcore_barrier()

    ebase = s * _EDGES_PER_TILE

    def _process(off, rem):
        if rem:
            pltpu.sync_copy(col_hbm.at[pl.ds(off, _REM)], colb.at[pl.ds(0, _REM)])
            pltpu.sync_copy(row_hbm.at[pl.ds(off, _REM)], rowb.at[pl.ds(0, _REM)])
            pltpu.sync_copy(val_hbm.at[pl.ds(off, _REM)], valb.at[pl.ds(0, _REM)])
            for g in range(_REM // 16, _CHUNK // 16):
                valb[pl.ds(g * 16, 16)] = zero16
        else:
            pltpu.sync_copy(col_hbm.at[pl.ds(off, _CHUNK)], colb)
            pltpu.sync_copy(row_hbm.at[pl.ds(off, _CHUNK)], rowb)
            pltpu.sync_copy(val_hbm.at[pl.ds(off, _CHUNK)], valb)

        for g in range(_CHUNK // 16):
            r = rowb[pl.ds(g * 16, 16)]
            local = r - lo
            ok = (local >= 0) & (local < _HALF)
            idxb[pl.ds(g * 16, 16)] = jnp.where(ok, local, dump)

        pltpu.async_copy(x_hbm.at[colb], rowsb, sem).wait()

        def _scale(g, carry):
            vv = valb[pl.ds(g * 16, 16)]
            for i in range(16):
                e = g * 16 + i
                v = vv[i]
                for q in range(_EMB // 16):
                    rowsb[e, pl.ds(q * 16, 16)] = rowsb[e, pl.ds(q * 16, 16)] * v
            return carry

        lax.fori_loop(0, _CHUNK // 16, _scale, 0)

        pltpu.sync_copy(rowsb, acc.at[idxb], add=True)

    def _chunk_body(j, carry):
        _process(ebase + j * _CHUNK, False)
        return carry

    lax.fori_loop(0, _NFULL, _chunk_body, 0)
    _process(ebase + _NFULL * _CHUNK, True)

    plsc.subcore_barrier()

    start = jnp.minimum(s * _CP_ROWS, _HALF - _CP_ROWS)
    pltpu.sync_copy(acc.at[pl.ds(start, _CP_ROWS)],
                    y_hbm.at[pl.ds(lo + start, _CP_ROWS)])


def _mean_body(a_ref, b_ref, c_ref, o_ref):
    o_ref[...] = (a_ref[...] + b_ref[...] + c_ref[...]) * (1.0 / 3.0)


_R2 = _N * _EMB // 128
_BLK = 1000
_mean3 = pl.pallas_call(
    _mean_body,
    grid=(_R2 // _BLK,),
    in_specs=[pl.BlockSpec((_BLK, 128), lambda i: (i, 0))] * 3,
    out_specs=pl.BlockSpec((_BLK, 128), lambda i: (i, 0)),
    out_shape=jax.ShapeDtypeStruct((_R2, 128), jnp.float32),
)


def kernel(user_emb, item_emb, adj_val, adj_row, adj_col):
    x0 = jnp.concatenate([user_emb, item_emb], axis=0)
    y1 = _spmm(x0, adj_row, adj_col, adj_val)
    y2 = _spmm(y1, adj_row, adj_col, adj_val)
    y3 = _spmm(y2, adj_row, adj_col, adj_val)
    m = _mean3(y1.reshape(_R2, 128), y2.reshape(_R2, 128), y3.reshape(_R2, 128))
    m = m.reshape(_N, _EMB)
    return m[:_USER_NUM], m[_USER_NUM:]


# speedup vs baseline: 4.4610x; 1.5646x over previous
import functools
import jax
import jax.numpy as jnp
from jax import lax
from jax.experimental import pallas as pl
from jax.experimental.pallas import tpu as pltpu, tpu_sc as plsc

_USER_NUM = 20000
_ITEM_NUM = 30000
_N = _USER_NUM + _ITEM_NUM
_E = 800000
_EMB = 64

_HALF = _N // 2
_NTILES = 16
_SUB = 128
_MACRO = 128
_NSUB = _MACRO // _SUB
_EDGES_PER_TILE = _E // _NTILES
_NMACRO = -(-_EDGES_PER_TILE // _MACRO)
_LAST = _EDGES_PER_TILE - (_NMACRO - 1) * _MACRO
_NBUF = 3
_ACC_ROWS = 25088
_ZROWS = _ACC_ROWS // _NTILES
_CP_ROWS = 1568

_mesh = plsc.VectorSubcoreMesh(core_axis_name="c", subcore_axis_name="s")

_scratch = []
for _ in range(_NBUF):
    _scratch += [
        pltpu.VMEM((_MACRO,), jnp.int32),
        pltpu.VMEM((_MACRO,), jnp.int32),
        pltpu.VMEM((_MACRO,), jnp.float32),
        pltpu.VMEM((_NSUB, _SUB), jnp.int32),
        pltpu.VMEM((_MACRO, _EMB), jnp.float32),
        pltpu.SemaphoreType.DMA,
        pltpu.SemaphoreType.DMA,
    ]
_scratch.append(pltpu.VMEM_SHARED((_ACC_ROWS, _EMB), jnp.float32))


@functools.partial(
    pl.kernel,
    mesh=_mesh,
    compiler_params=pltpu.CompilerParams(use_tc_tiling_on_sc=False),
    out_type=jax.ShapeDtypeStruct((_N, _EMB), jnp.float32),
    scratch_types=_scratch,
)
def _spmm(x_hbm, row_hbm, col_hbm, val_hbm, y_hbm, *rest):
    bufs = []
    for b in range(_NBUF):
        bufs.append(rest[b * 7:(b + 1) * 7])
    acc = rest[_NBUF * 7]

    c = lax.axis_index("c")
    s = lax.axis_index("s")
    lo = c * _HALF
    dump = _HALF + s
    zero16 = jnp.zeros((16,), jnp.float32)
    ebase = s * _EDGES_PER_TILE

    rowsb0 = bufs[0][4]

    def _zero_rowsb(r, carry):
        for q in range(_EMB // 16):
            rowsb0[r, pl.ds(q * 16, 16)] = zero16
        return carry

    lax.fori_loop(0, _SUB, _zero_rowsb, 0)
    zbase = s * _ZROWS
    for k in range(_ZROWS // _SUB):
        pltpu.sync_copy(rowsb0.at[pl.ds(0, _SUB)], acc.at[pl.ds(zbase + k * _SUB, _SUB)])
    pltpu.sync_copy(rowsb0.at[pl.ds(0, _SUB)],
                    acc.at[pl.ds(zbase + _ZROWS - _SUB, _SUB)])
    plsc.subcore_barrier()

    def _load(m, b):
        colb, rowb, valb = bufs[b][0], bufs[b][1], bufs[b][2]
        off = ebase + m * _MACRO
        pltpu.sync_copy(col_hbm.at[pl.ds(off, _MACRO)], colb)
        pltpu.sync_copy(row_hbm.at[pl.ds(off, _MACRO)], rowb)
        pltpu.sync_copy(val_hbm.at[pl.ds(off, _MACRO)], valb)

    def _load_last(b):
        colb, rowb, valb = bufs[b][0], bufs[b][1], bufs[b][2]
        off = ebase + (_NMACRO - 1) * _MACRO
        pltpu.sync_copy(col_hbm.at[pl.ds(off, _LAST)], colb.at[pl.ds(0, _LAST)])
        pltpu.sync_copy(row_hbm.at[pl.ds(off, _LAST)], rowb.at[pl.ds(0, _LAST)])
        pltpu.sync_copy(val_hbm.at[pl.ds(off, _LAST)], valb.at[pl.ds(0, _LAST)])
        for t in range(_LAST // 16, _MACRO // 16):
            valb[pl.ds(t * 16, 16)] = zero16

    def _gather_start(b):
        colb, rowsb, gsem = bufs[b][0], bufs[b][4], bufs[b][5]
        for j in range(_NSUB):
            pltpu.async_copy(x_hbm.at[colb.at[pl.ds(j * _SUB, _SUB)]],
                             rowsb.at[pl.ds(j * _SUB, _SUB)], gsem)

    def _gather_wait(b):
        colb, rowsb, gsem = bufs[b][0], bufs[b][4], bufs[b][5]
        for j in range(_NSUB):
            pltpu.make_async_copy(x_hbm.at[colb.at[pl.ds(j * _SUB, _SUB)]],
                                  rowsb.at[pl.ds(j * _SUB, _SUB)], gsem).wait()

    def _index(b):
        rowb, idxb = bufs[b][1], bufs[b][3]
        for j in range(_NSUB):
            for g in range(_SUB // 16):
                r = rowb[pl.ds(j * _SUB + g * 16, 16)]
                local = r - lo
                ok = (local >= 0) & (local < _HALF)
                idxb[j, pl.ds(g * 16, 16)] = jnp.where(ok, local, dump)

    def _scale(b):
        valb, rowsb = bufs[b][2], bufs[b][4]

        def _grp(g, carry):
            vv = valb[pl.ds(g * 16, 16)]
            for i in range(16):
                e = g * 16 + i
                v = vv[i]
                for q in range(_EMB // 16):
                    rowsb[e, pl.ds(q * 16, 16)] = rowsb[e, pl.ds(q * 16, 16)] * v
            return carry

        lax.fori_loop(0, _MACRO // 16, _grp, 0)

    def _scatter_start(b):
        idxb, rowsb, ssem = bufs[b][3], bufs[b][4], bufs[b][6]
        for j in range(_NSUB):
            pltpu.async_copy(rowsb.at[pl.ds(j * _SUB, _SUB)],
                             acc.at[idxb.at[j]], ssem, add=True)

    def _scatter_wait(b):
        idxb, rowsb, ssem = bufs[b][3], bufs[b][4], bufs[b][6]
        for j in range(_NSUB):
            pltpu.make_async_copy(rowsb.at[pl.ds(j * _SUB, _SUB)],
                                  acc.at[idxb.at[j]], ssem).wait()

    _load(jnp.int32(0), 0)
    _gather_start(0)

    def _body(k, carry):
        for i in range(_NBUF):
            b = i
            nb = (i + 1) % _NBUF
            m = k * _NBUF + i
            if i == _NBUF - 1:
                _scatter_wait(nb)
            else:
                @pl.when(k > 0)
                def _():
                    _scatter_wait(nb)
            _load(m + 1, nb)
            _gather_start(nb)
            _index(b)
            _gather_wait(b)
            _scale(b)
            _scatter_start(b)
        return carry

    _NLOOP = (_NMACRO - 4) // _NBUF
    lax.fori_loop(0, _NLOOP, _body, 0)

    def _step(m, b, nb, last=False):
        _scatter_wait(nb)
        if last:
            _load_last(nb)
        else:
            _load(m + 1, nb)
        _gather_start(nb)
        _index(b)
        _gather_wait(b)
        _scale(b)
        _scatter_start(b)

    m0 = _NLOOP * _NBUF
    _step(m0, 0, 1)
    _step(m0 + 1, 1, 2)
    _step(m0 + 2, 2, 0, last=True)
    _index(0)
    _gather_wait(0)
    _scale(0)
    _scatter_start(0)

    _scatter_wait(1)
    _scatter_wait(2)
    _scatter_wait(0)

    plsc.subcore_barrier()

    start = jnp.minimum(s * _CP_ROWS, _HALF - _CP_ROWS)
    pltpu.sync_copy(acc.at[pl.ds(start, _CP_ROWS)],
                    y_hbm.at[pl.ds(lo + start, _CP_ROWS)])


def _mean_body(a_ref, b_ref, c_ref, o_ref):
    o_ref[...] = (a_ref[...] + b_ref[...] + c_ref[...]) * (1.0 / 3.0)


_R2 = _N * _EMB // 128
_BLK = 1000
_mean3 = pl.pallas_call(
    _mean_body,
    grid=(_R2 // _BLK,),
    in_specs=[pl.BlockSpec((_BLK, 128), lambda i: (i, 0))] * 3,
    out_specs=pl.BlockSpec((_BLK, 128), lambda i: (i, 0)),
    out_shape=jax.ShapeDtypeStruct((_R2, 128), jnp.float32),
)


def kernel(user_emb, item_emb, adj_val, adj_row, adj_col):
    x0 = jnp.concatenate([user_emb, item_emb], axis=0)
    y1 = _spmm(x0, adj_row, adj_col, adj_val)
    y2 = _spmm(y1, adj_row, adj_col, adj_val)
    y3 = _spmm(y2, adj_row, adj_col, adj_val)
    m = _mean3(y1.reshape(_R2, 128), y2.reshape(_R2, 128), y3.reshape(_R2, 128))
    m = m.reshape(_N, _EMB)
    return m[:_USER_NUM], m[_USER_NUM:]
